# baseline (speedup 1.0000x reference)
import functools

import numpy as np
import jax
import jax.numpy as jnp
from jax import lax
from jax.experimental import pallas as pl
from jax.experimental.pallas import tpu as pltpu
from jax.experimental.pallas import tpu_sc as plsc

N = 10000
D = 128
R = 24

NC = 2
NS = 16
NW = NC * NS

CH = 128
TRASH = 16
ROWS_PT = 632
ACC_ROWS = ROWS_PT * NS

BN = 1000


def _l1_body(h_ref, w_ref, o_ref):
    o_ref[...] = jnp.dot(h_ref[...], w_ref[0],
                         preferred_element_type=jnp.float32)[None, :, :]


def _finish_body(p_ref, dg_ref, o_ref):
    acc = p_ref[0] + p_ref[1]
    deg = jnp.maximum(dg_ref[0, :, 0:1] + dg_ref[1, :, 0:1], 1.0)
    o_ref[...] = jnp.maximum(acc / deg, 0.0)


def _transform_l1(x, W):
    return pl.pallas_call(
        _l1_body,
        grid=(N // BN, R),
        in_specs=[
            pl.BlockSpec((BN, D), lambda i, r: (i, 0)),
            pl.BlockSpec((1, D, D), lambda i, r: (r, 0, 0)),
        ],
        out_specs=pl.BlockSpec((1, BN, D), lambda i, r: (r, i, 0)),
        out_shape=jax.ShapeDtypeStruct((R, N, D), jnp.float32),
    )(x, W)


def _finish(p, dg):
    return pl.pallas_call(
        _finish_body,
        grid=(N // BN,),
        in_specs=[
            pl.BlockSpec((NC, BN, D), lambda i: (0, i, 0)),
            pl.BlockSpec((NC, BN, D), lambda i: (0, i, 0)),
        ],
        out_specs=pl.BlockSpec((BN, D), lambda i: (i, 0)),
        out_shape=jax.ShapeDtypeStruct((N, D), jnp.float32),
    )(p, dg)


def _sc_pass(ht, idxp, zrow, *, n_chunks):
    mesh = plsc.VectorSubcoreMesh(core_axis_name="c", subcore_axis_name="s")

    @functools.partial(
        pl.kernel, mesh=mesh,
        out_type=jax.ShapeDtypeStruct((NC, ACC_ROWS, D), jnp.float32),
        scratch_types=[
            pltpu.VMEM((2, CH), jnp.int32),
            pltpu.VMEM((2, CH), jnp.int32),
            pltpu.VMEM((CH, D), jnp.float32),
            pltpu.VMEM((CH, D), jnp.float32),
            pltpu.VMEM_SHARED((ACC_ROWS, D), jnp.float32),
            pltpu.SemaphoreType.DMA,
            pltpu.SemaphoreType.DMA,
            pltpu.SemaphoreType.DMA,
            pltpu.SemaphoreType.DMA,
        ])
    def k(ht_hbm, idx_hbm, zrow_hbm, out_hbm,
          pa, pb, rows_a, rows_b, acc, sem_a, sem_b, sem_ia, sem_ib):
        c = lax.axis_index("c")
        s = lax.axis_index("s")
        w = c * NS + s
        base = w * n_chunks

        pltpu.sync_copy(zrow_hbm, acc.at[pl.ds(s * ROWS_PT, ROWS_PT)])
        plsc.subcore_barrier()
        pltpu.sync_copy(idx_hbm.at[base], pa)
        pltpu.async_copy(ht_hbm.at[pa.at[0]], rows_a, sem_a)

        @pl.when(n_chunks > 1)
        def _():
            pltpu.async_copy(idx_hbm.at[base + 1], pb, sem_ib)

        def chunk(i, carry):
            @pl.when(i % 2 == 0)
            def _():
                pltpu.make_async_copy(ht_hbm.at[pa.at[0]], rows_a, sem_a).wait()

                @pl.when(i + 1 < n_chunks)
                def _():
                    pltpu.make_async_copy(idx_hbm.at[base], pb, sem_ib).wait()
                    pltpu.async_copy(ht_hbm.at[pb.at[0]], rows_b, sem_b)
                pltpu.sync_copy(rows_a, acc.at[pa.at[1]], add=True)

                @pl.when(i + 2 < n_chunks)
                def _():
                    pltpu.async_copy(idx_hbm.at[base + i + 2], pa, sem_ia)

            @pl.when(i % 2 == 1)
            def _():
                pltpu.make_async_copy(ht_hbm.at[pb.at[0]], rows_b, sem_b).wait()

                @pl.when(i + 1 < n_chunks)
                def _():
                    pltpu.make_async_copy(idx_hbm.at[base], pa, sem_ia).wait()
                    pltpu.async_copy(ht_hbm.at[pa.at[0]], rows_a, sem_a)
                pltpu.sync_copy(rows_b, acc.at[pb.at[1]], add=True)

                @pl.when(i + 2 < n_chunks)
                def _():
                    pltpu.async_copy(idx_hbm.at[base + i + 2], pb, sem_ib)
            return carry

        lax.fori_loop(0, n_chunks, chunk, 0)
        plsc.subcore_barrier()

        sl = pl.ds(s * ROWS_PT, ROWS_PT)
        pltpu.sync_copy(acc.at[sl], out_hbm.at[c, sl])

    return k(ht, idxp, zrow)


def _deg_pass(dstp, zrow, ones, *, n_chunks):
    mesh = plsc.VectorSubcoreMesh(core_axis_name="c", subcore_axis_name="s")

    @functools.partial(
        pl.kernel, mesh=mesh,
        out_type=jax.ShapeDtypeStruct((NC, ACC_ROWS, D), jnp.float32),
        scratch_types=[
            pltpu.VMEM((n_chunks, CH), jnp.int32),
            pltpu.VMEM((CH, D), jnp.float32),
            pltpu.VMEM_SHARED((ACC_ROWS, D), jnp.float32),
        ])
    def k(dst_hbm, zrow_hbm, ones_hbm, out_hbm, dstv, onesv, acc):
        c = lax.axis_index("c")
        s = lax.axis_index("s")
        w = c * NS + s

        pltpu.sync_copy(dst_hbm.at[pl.ds(w * n_chunks, n_chunks)], dstv)
        pltpu.sync_copy(zrow_hbm, acc.at[pl.ds(s * ROWS_PT, ROWS_PT)])
        pltpu.sync_copy(ones_hbm, onesv)
        plsc.subcore_barrier()

        def chunk(i, carry):
            pltpu.sync_copy(onesv, acc.at[dstv.at[i]], add=True)
            return carry

        lax.fori_loop(0, n_chunks, chunk, 0)
        plsc.subcore_barrier()

        sl = pl.ds(s * ROWS_PT, ROWS_PT)
        pltpu.sync_copy(acc.at[sl], out_hbm.at[c, sl])

    return k(dstp, zrow, ones)


def kernel(x, edge_index, edge_type, W0, W1):
    E = edge_index.shape[1]
    n_chunks = -(-E // (NW * CH))
    n_chunks = (n_chunks + 7) // 8 * 8
    e_pad = n_chunks * CH * NW
    npad = e_pad - E
    pad_src = jnp.asarray(np.arange(npad, dtype=np.int32) % N)
    pad_typ = jnp.zeros((npad,), jnp.int32)
    pad_dst = jnp.asarray(N + (np.arange(npad, dtype=np.int32) % TRASH))

    srcp = jnp.concatenate([edge_index[0], pad_src])
    typp = jnp.concatenate([edge_type, pad_typ])
    gidp = (typp * N + srcp).reshape(NW * n_chunks, CH)
    dstp = jnp.concatenate([edge_index[1], pad_dst]).reshape(NW * n_chunks, CH)
    idxp = jnp.stack([gidp, dstp], axis=1)

    zrow = jnp.zeros((ROWS_PT, D), jnp.float32)
    ones = jnp.ones((CH, D), jnp.float32)

    dg = _deg_pass(dstp, zrow, ones, n_chunks=n_chunks)
    ht0 = _transform_l1(x, W0).reshape(R * N, D)
    p0 = _sc_pass(ht0, idxp, zrow, n_chunks=n_chunks)
    h1 = _finish(p0, dg)
    ht1 = _transform_l1(h1, W1).reshape(R * N, D)
    p1 = _sc_pass(ht1, idxp, zrow, n_chunks=n_chunks)
    return _finish(p1, dg)


# speedup vs baseline: 1.1269x; 1.1269x over previous
import functools

import numpy as np
import jax
import jax.numpy as jnp
from jax import lax
from jax.experimental import pallas as pl
from jax.experimental.pallas import tpu as pltpu
from jax.experimental.pallas import tpu_sc as plsc

N = 10000
D = 128
R = 24

NC = 2
NS = 16
NW = NC * NS

CH = 128
TRASH = 16
ROWS_PT = 632
ACC_ROWS = ROWS_PT * NS

BN = 2000


def _l1_body(h_ref, w_ref, o_ref):
    o_ref[...] = jnp.dot(h_ref[...], w_ref[0],
                         preferred_element_type=jnp.float32)[None, :, :]


def _finish_body(p_ref, dg_ref, o_ref):
    acc = p_ref[0] + p_ref[1]
    deg = jnp.maximum(dg_ref[0, :, 0:1] + dg_ref[1, :, 0:1], 1.0)
    o_ref[...] = jnp.maximum(acc / deg, 0.0)


def _transform_l1(x, W):
    return pl.pallas_call(
        _l1_body,
        grid=(N // BN, R),
        in_specs=[
            pl.BlockSpec((BN, D), lambda i, r: (i, 0)),
            pl.BlockSpec((1, D, D), lambda i, r: (r, 0, 0)),
        ],
        out_specs=pl.BlockSpec((1, BN, D), lambda i, r: (r, i, 0)),
        out_shape=jax.ShapeDtypeStruct((R, N, D), jnp.float32),
    )(x, W)


def _finish(p, dg):
    return pl.pallas_call(
        _finish_body,
        grid=(N // BN,),
        in_specs=[
            pl.BlockSpec((NC, BN, D), lambda i: (0, i, 0)),
            pl.BlockSpec((NC, BN, D), lambda i: (0, i, 0)),
        ],
        out_specs=pl.BlockSpec((BN, D), lambda i: (i, 0)),
        out_shape=jax.ShapeDtypeStruct((N, D), jnp.float32),
    )(p, dg)


def _sc_pass(ht, idxp, zrow, ones, *, n_chunks, with_deg):
    mesh = plsc.VectorSubcoreMesh(core_axis_name="c", subcore_axis_name="s")
    out_type = [jax.ShapeDtypeStruct((NC, ACC_ROWS, D), jnp.float32)]
    if with_deg:
        out_type.append(jax.ShapeDtypeStruct((NC, ACC_ROWS, D), jnp.float32))

    @functools.partial(
        pl.kernel, mesh=mesh,
        out_type=tuple(out_type),
        scratch_types=[
            pltpu.VMEM((2, CH), jnp.int32),
            pltpu.VMEM((2, CH), jnp.int32),
            pltpu.VMEM((CH, D), jnp.float32),
            pltpu.VMEM((CH, D), jnp.float32),
            pltpu.VMEM_SHARED((ACC_ROWS, D), jnp.float32),
            pltpu.SemaphoreType.DMA,
            pltpu.SemaphoreType.DMA,
            pltpu.SemaphoreType.DMA,
            pltpu.SemaphoreType.DMA,
        ])
    def k(ht_hbm, idx_hbm, zrow_hbm, ones_hbm, *refs):
        if with_deg:
            out_hbm, deg_hbm = refs[:2]
            refs = refs[2:]
        else:
            out_hbm = refs[0]
            refs = refs[1:]
        pa, pb, rows_a, rows_b, acc, sem_a, sem_b, sem_ia, sem_ib = refs
        c = lax.axis_index("c")
        s = lax.axis_index("s")
        w = c * NS + s
        base = w * n_chunks
        tsl = pl.ds(s * ROWS_PT, ROWS_PT)
        csl = pl.ds(s * ROWS_PT, ROWS_PT)

        pltpu.sync_copy(zrow_hbm, acc.at[tsl])
        plsc.subcore_barrier()

        if with_deg:
            pltpu.sync_copy(ones_hbm, rows_a)
            pltpu.sync_copy(idx_hbm.at[base], pa)

            @pl.when(n_chunks > 1)
            def _():
                pltpu.async_copy(idx_hbm.at[base + 1], pb, sem_ib)

            def dchunk(i, carry):
                @pl.when(i % 2 == 0)
                def _():
                    @pl.when(i > 0)
                    def _():
                        pltpu.make_async_copy(idx_hbm.at[base], pa, sem_ia).wait()
                    pltpu.sync_copy(rows_a, acc.at[pa.at[1]], add=True)

                    @pl.when(i + 2 < n_chunks)
                    def _():
                        pltpu.async_copy(idx_hbm.at[base + i + 2], pa, sem_ia)

                @pl.when(i % 2 == 1)
                def _():
                    pltpu.make_async_copy(idx_hbm.at[base], pb, sem_ib).wait()
                    pltpu.sync_copy(rows_a, acc.at[pb.at[1]], add=True)

                    @pl.when(i + 2 < n_chunks)
                    def _():
                        pltpu.async_copy(idx_hbm.at[base + i + 2], pb, sem_ib)
                return carry

            lax.fori_loop(0, n_chunks, dchunk, 0)
            plsc.subcore_barrier()
            pltpu.sync_copy(acc.at[csl], deg_hbm.at[c, csl])
            pltpu.sync_copy(zrow_hbm, acc.at[tsl])
            plsc.subcore_barrier()

        pltpu.sync_copy(idx_hbm.at[base], pa)
        pltpu.async_copy(ht_hbm.at[pa.at[0]], rows_a, sem_a)

        @pl.when(n_chunks > 1)
        def _():
            pltpu.async_copy(idx_hbm.at[base + 1], pb, sem_ib)

        def chunk(i, carry):
            @pl.when(i % 2 == 0)
            def _():
                pltpu.make_async_copy(ht_hbm.at[pa.at[0]], rows_a, sem_a).wait()

                @pl.when(i + 1 < n_chunks)
                def _():
                    pltpu.make_async_copy(idx_hbm.at[base], pb, sem_ib).wait()
                    pltpu.async_copy(ht_hbm.at[pb.at[0]], rows_b, sem_b)
                pltpu.sync_copy(rows_a, acc.at[pa.at[1]], add=True)

                @pl.when(i + 2 < n_chunks)
                def _():
                    pltpu.async_copy(idx_hbm.at[base + i + 2], pa, sem_ia)

            @pl.when(i % 2 == 1)
            def _():
                pltpu.make_async_copy(ht_hbm.at[pb.at[0]], rows_b, sem_b).wait()

                @pl.when(i + 1 < n_chunks)
                def _():
                    pltpu.make_async_copy(idx_hbm.at[base], pa, sem_ia).wait()
                    pltpu.async_copy(ht_hbm.at[pa.at[0]], rows_a, sem_a)
                pltpu.sync_copy(rows_b, acc.at[pb.at[1]], add=True)

                @pl.when(i + 2 < n_chunks)
                def _():
                    pltpu.async_copy(idx_hbm.at[base + i + 2], pb, sem_ib)
            return carry

        lax.fori_loop(0, n_chunks, chunk, 0)
        plsc.subcore_barrier()

        pltpu.sync_copy(acc.at[csl], out_hbm.at[c, csl])

    return k(ht, idxp, zrow, ones)


def kernel(x, edge_index, edge_type, W0, W1):
    E = edge_index.shape[1]
    n_chunks = -(-E // (NW * CH))
    n_chunks = (n_chunks + 7) // 8 * 8
    e_pad = n_chunks * CH * NW
    npad = e_pad - E
    pad_src = jnp.asarray(np.arange(npad, dtype=np.int32) % N)
    pad_typ = jnp.zeros((npad,), jnp.int32)
    pad_dst = jnp.asarray(N + (np.arange(npad, dtype=np.int32) % TRASH))

    srcp = jnp.concatenate([edge_index[0], pad_src])
    typp = jnp.concatenate([edge_type, pad_typ])
    gidp = (typp * N + srcp).reshape(NW * n_chunks, CH)
    dstp = jnp.concatenate([edge_index[1], pad_dst]).reshape(NW * n_chunks, CH)
    idxp = jnp.stack([gidp, dstp], axis=1)

    zrow = jnp.zeros((ROWS_PT, D), jnp.float32)
    ones = jnp.ones((CH, D), jnp.float32)

    ht0 = _transform_l1(x, W0).reshape(R * N, D)
    p0, dg = _sc_pass(ht0, idxp, zrow, ones, n_chunks=n_chunks, with_deg=True)
    h1 = _finish(p0, dg)
    ht1 = _transform_l1(h1, W1).reshape(R * N, D)
    out1 = _sc_pass(ht1, idxp, zrow, ones, n_chunks=n_chunks, with_deg=False)
    p1 = out1[0] if isinstance(out1, (tuple, list)) else out1
    return _finish(p1, dg)


# speedup vs baseline: 1.1302x; 1.0029x over previous
import functools

import numpy as np
import jax
import jax.numpy as jnp
from jax import lax
from jax.experimental import pallas as pl
from jax.experimental.pallas import tpu as pltpu
from jax.experimental.pallas import tpu_sc as plsc

N = 10000
D = 128
R = 24

NC = 2
NS = 16
NW = NC * NS

CH = 128
TRASH = 16
ROWS_PT = 632
ACC_ROWS = ROWS_PT * NS

BN = 2000


def _l1_body(h_ref, w_ref, o_ref):
    o_ref[...] = jnp.dot(h_ref[...], w_ref[0],
                         preferred_element_type=jnp.float32)[None, :, :]


def _finish_body(p_ref, dg_ref, o_ref):
    acc = p_ref[0] + p_ref[1]
    deg = jnp.maximum(dg_ref[0, :, 0:1] + dg_ref[1, :, 0:1], 1.0)
    o_ref[...] = jnp.maximum(acc / deg, 0.0)


def _transform_l1(x, W):
    return pl.pallas_call(
        _l1_body,
        grid=(N // BN, R),
        in_specs=[
            pl.BlockSpec((BN, D), lambda i, r: (i, 0)),
            pl.BlockSpec((1, D, D), lambda i, r: (r, 0, 0)),
        ],
        out_specs=pl.BlockSpec((1, BN, D), lambda i, r: (r, i, 0)),
        out_shape=jax.ShapeDtypeStruct((R, N, D), jnp.float32),
    )(x, W)


def _finish(p, dg):
    return pl.pallas_call(
        _finish_body,
        grid=(N // BN,),
        in_specs=[
            pl.BlockSpec((NC, BN, D), lambda i: (0, i, 0)),
            pl.BlockSpec((NC, BN, D), lambda i: (0, i, 0)),
        ],
        out_specs=pl.BlockSpec((BN, D), lambda i: (i, 0)),
        out_shape=jax.ShapeDtypeStruct((N, D), jnp.float32),
    )(p, dg)


def _sc_pass(ht, idxp, zrow, ones, *, n_chunks, with_deg):
    mesh = plsc.VectorSubcoreMesh(core_axis_name="c", subcore_axis_name="s")
    out_type = [jax.ShapeDtypeStruct((NC, ACC_ROWS, D), jnp.float32)]
    if with_deg:
        out_type.append(jax.ShapeDtypeStruct((NC, ACC_ROWS, D), jnp.float32))

    @functools.partial(
        pl.kernel, mesh=mesh,
        out_type=tuple(out_type),
        scratch_types=[
            pltpu.VMEM((2, CH), jnp.int32),
            pltpu.VMEM((2, CH), jnp.int32),
            pltpu.VMEM((CH, D), jnp.float32),
            pltpu.VMEM((CH, D), jnp.float32),
            pltpu.VMEM_SHARED((ACC_ROWS, D), jnp.float32),
            pltpu.SemaphoreType.DMA,
            pltpu.SemaphoreType.DMA,
            pltpu.SemaphoreType.DMA,
            pltpu.SemaphoreType.DMA,
        ])
    def k(ht_hbm, idx_hbm, zrow_hbm, ones_hbm, *refs):
        if with_deg:
            out_hbm, deg_hbm = refs[:2]
            refs = refs[2:]
        else:
            out_hbm = refs[0]
            refs = refs[1:]
        pa, pb, rows_a, rows_b, acc, sem_a, sem_b, sem_ia, sem_ib = refs
        c = lax.axis_index("c")
        s = lax.axis_index("s")
        w = c * NS + s
        base = w * n_chunks
        tsl = pl.ds(s * ROWS_PT, ROWS_PT)
        csl = pl.ds(s * ROWS_PT, ROWS_PT)

        pltpu.sync_copy(zrow_hbm, acc.at[tsl])
        plsc.subcore_barrier()

        if with_deg:
            pltpu.sync_copy(ones_hbm, rows_a)
            pltpu.sync_copy(idx_hbm.at[base], pa)

            @pl.when(n_chunks > 1)
            def _():
                pltpu.async_copy(idx_hbm.at[base + 1], pb, sem_ib)

            def dchunk(i, carry):
                @pl.when(i % 2 == 0)
                def _():
                    @pl.when(i > 0)
                    def _():
                        pltpu.make_async_copy(idx_hbm.at[base], pa, sem_ia).wait()
                    pltpu.async_copy(rows_a, acc.at[pa.at[1]], sem_a, add=True)

                    @pl.when(i > 0)
                    def _():
                        pltpu.make_async_copy(
                            rows_a, acc.at[pb.at[1]], sem_b).wait()

                        @pl.when(i + 1 < n_chunks)
                        def _():
                            pltpu.async_copy(idx_hbm.at[base + i + 1], pb, sem_ib)

                @pl.when(i % 2 == 1)
                def _():
                    pltpu.make_async_copy(idx_hbm.at[base], pb, sem_ib).wait()
                    pltpu.async_copy(rows_a, acc.at[pb.at[1]], sem_b, add=True)
                    pltpu.make_async_copy(rows_a, acc.at[pa.at[1]], sem_a).wait()

                    @pl.when(i + 1 < n_chunks)
                    def _():
                        pltpu.async_copy(idx_hbm.at[base + i + 1], pa, sem_ia)
                return carry

            lax.fori_loop(0, n_chunks, dchunk, 0)
            pltpu.make_async_copy(rows_a, acc.at[pb.at[1]], sem_b).wait()
            plsc.subcore_barrier()
            pltpu.sync_copy(acc.at[csl], deg_hbm.at[c, csl])
            pltpu.sync_copy(zrow_hbm, acc.at[tsl])
            plsc.subcore_barrier()

        pltpu.sync_copy(idx_hbm.at[base], pa)
        pltpu.async_copy(ht_hbm.at[pa.at[0]], rows_a, sem_a)

        @pl.when(n_chunks > 1)
        def _():
            pltpu.async_copy(idx_hbm.at[base + 1], pb, sem_ib)

        def chunk(i, carry):
            @pl.when(i % 2 == 0)
            def _():
                pltpu.make_async_copy(ht_hbm.at[pa.at[0]], rows_a, sem_a).wait()

                @pl.when(i + 1 < n_chunks)
                def _():
                    pltpu.make_async_copy(idx_hbm.at[base], pb, sem_ib).wait()
                    pltpu.async_copy(ht_hbm.at[pb.at[0]], rows_b, sem_b)
                pltpu.sync_copy(rows_a, acc.at[pa.at[1]], add=True)

                @pl.when(i + 2 < n_chunks)
                def _():
                    pltpu.async_copy(idx_hbm.at[base + i + 2], pa, sem_ia)

            @pl.when(i % 2 == 1)
            def _():
                pltpu.make_async_copy(ht_hbm.at[pb.at[0]], rows_b, sem_b).wait()

                @pl.when(i + 1 < n_chunks)
                def _():
                    pltpu.make_async_copy(idx_hbm.at[base], pa, sem_ia).wait()
                    pltpu.async_copy(ht_hbm.at[pa.at[0]], rows_a, sem_a)
                pltpu.sync_copy(rows_b, acc.at[pb.at[1]], add=True)

                @pl.when(i + 2 < n_chunks)
                def _():
                    pltpu.async_copy(idx_hbm.at[base + i + 2], pb, sem_ib)
            return carry

        lax.fori_loop(0, n_chunks, chunk, 0)
        plsc.subcore_barrier()

        pltpu.sync_copy(acc.at[csl], out_hbm.at[c, csl])

    return k(ht, idxp, zrow, ones)


def kernel(x, edge_index, edge_type, W0, W1):
    E = edge_index.shape[1]
    n_chunks = -(-E // (NW * CH))
    n_chunks = (n_chunks + 7) // 8 * 8
    e_pad = n_chunks * CH * NW
    npad = e_pad - E
    pad_src = jnp.asarray(np.arange(npad, dtype=np.int32) % N)
    pad_typ = jnp.zeros((npad,), jnp.int32)
    pad_dst = jnp.asarray(N + (np.arange(npad, dtype=np.int32) % TRASH))

    srcp = jnp.concatenate([edge_index[0], pad_src])
    typp = jnp.concatenate([edge_type, pad_typ])
    gidp = (typp * N + srcp).reshape(NW * n_chunks, CH)
    dstp = jnp.concatenate([edge_index[1], pad_dst]).reshape(NW * n_chunks, CH)
    idxp = jnp.stack([gidp, dstp], axis=1)

    zrow = jnp.zeros((ROWS_PT, D), jnp.float32)
    ones = jnp.ones((CH, D), jnp.float32)

    ht0 = _transform_l1(x, W0).reshape(R * N, D)
    p0, dg = _sc_pass(ht0, idxp, zrow, ones, n_chunks=n_chunks, with_deg=True)
    h1 = _finish(p0, dg)
    ht1 = _transform_l1(h1, W1).reshape(R * N, D)
    out1 = _sc_pass(ht1, idxp, zrow, ones, n_chunks=n_chunks, with_deg=False)
    p1 = out1[0] if isinstance(out1, (tuple, list)) else out1
    return _finish(p1, dg)
